# baseline (speedup 1.0000x reference)
import functools

import jax
import jax.numpy as jnp
from jax import lax
from jax.experimental import pallas as pl
from jax.experimental.pallas import tpu as pltpu
from jax.experimental.pallas import tpu_sc as plsc

B, C_IN, C_OUT, N, K = 1, 128, 128, 10000, 32
DEG = K + 1

NSUB = 16
NPT_SLOW = 128
NPT_FAST = 512
SLOW_CID = 1
N_PAD = NSUB * (NPT_SLOW + NPT_FAST)
GROUP = 8
GROW = GROUP * K
NBUF = 2
LANES = 16
CVECS = C_OUT // LANES
GMAX = NPT_FAST // GROUP
TC_BLK = 1024


def _tc_body(x_ref, w_ref, b_ref, o_ref):
    acc = lax.dot_general(
        x_ref[:, :], w_ref[:, :],
        (((0,), (1,)), ((), ())),
        preferred_element_type=jnp.float32,
    )
    o_ref[:, :] = jnp.maximum(acc, 0.0) + b_ref[:, :]


def _compute_h(x_pad, w_scaled, b_scaled):
    return pl.pallas_call(
        _tc_body,
        grid=(N_PAD // TC_BLK,),
        in_specs=[
            pl.BlockSpec((C_IN, TC_BLK), lambda i: (0, i)),
            pl.BlockSpec((C_OUT, C_IN), lambda i: (0, 0)),
            pl.BlockSpec((1, C_OUT), lambda i: (0, 0)),
        ],
        out_specs=pl.BlockSpec((TC_BLK, C_OUT), lambda i: (i, 0)),
        out_shape=jax.ShapeDtypeStruct((N_PAD, C_OUT), jnp.float32),
    )(x_pad, w_scaled, b_scaled)


def _worker(h_hbm, ei_hbm, out_hbm, idx_v, rows0_v, rows1_v,
            self0_v, self1_v, out0_v, out1_v,
            gsem0, gsem1, ssem0, ssem1, osem0, osem1,
            node_base, ngroups):
    node_base = pl.multiple_of(node_base, 8)
    rowbase = pl.multiple_of(node_base // GROUP, 8)

    rows = (rows0_v, rows1_v)
    selfs = (self0_v, self1_v)
    outs = (out0_v, out1_v)
    gsems = (gsem0, gsem1)
    ssems = (ssem0, ssem1)
    osems = (osem0, osem1)

    pltpu.sync_copy(ei_hbm.at[pl.ds(rowbase * 2, GMAX * 2)], idx_v)

    HALF = GROW // 2

    def gather(g, b):
        pltpu.async_copy(h_hbm.at[idx_v.at[2 * g]],
                         rows[b].at[pl.ds(0, HALF)], gsems[b])
        pltpu.async_copy(h_hbm.at[idx_v.at[2 * g + 1]],
                         rows[b].at[pl.ds(HALF, HALF)], gsems[b])
        pltpu.async_copy(
            h_hbm.at[pl.ds(node_base + g * GROUP, GROUP)], selfs[b],
            ssems[b])

    def wait_gather(g, b):
        pltpu.make_async_copy(h_hbm.at[idx_v.at[2 * g]],
                              rows[b].at[pl.ds(0, HALF)],
                              gsems[b]).wait()
        pltpu.make_async_copy(h_hbm.at[idx_v.at[2 * g + 1]],
                              rows[b].at[pl.ds(HALF, HALF)],
                              gsems[b]).wait()
        pltpu.make_async_copy(
            h_hbm.at[pl.ds(node_base + g * GROUP, GROUP)], selfs[b],
            ssems[b]).wait()

    def out_dma(g, b):
        pltpu.async_copy(
            outs[b], out_hbm.at[pl.ds(node_base + g * GROUP, GROUP)],
            osems[b])

    def wait_out(g, b):
        pltpu.make_async_copy(
            outs[b], out_hbm.at[pl.ds(node_base + g * GROUP, GROUP)],
            osems[b]).wait()

    def consume(g, b):
        rows_b = rows[b]
        self_b = selfs[b]
        out_b = outs[b]

        def node(n_, carry):
            base = n_ * K
            for c in range(CVECS):
                cs = pl.ds(c * LANES, LANES)
                s = self_b[n_, cs]
                for r in range(K):
                    s = s + rows_b[base + r, cs]
                out_b[n_, cs] = s
            return carry

        lax.fori_loop(0, GROUP, node, 0)

    for b in range(NBUF):
        gather(b, b)

    def body(o, carry):
        for b in range(NBUF):
            g = o * NBUF + b
            wait_gather(g, b)

            @pl.when(o > 0)
            def _():
                wait_out(g - NBUF, b)

            consume(g, b)
            out_dma(g, b)
            gather(g + NBUF, b)
        return carry

    lax.fori_loop(0, ngroups // NBUF - 1, body, 0)

    g0 = ngroups - NBUF
    for b in range(NBUF):
        g = g0 + b
        wait_gather(g, b)

        @pl.when(g0 > 0)
        def _():
            wait_out(g - NBUF, b)

        consume(g, b)
        out_dma(g, b)
    for b in range(NBUF):
        wait_out(g0 + b, b)


def _sc_body(h_hbm, ei_hbm, out_hbm, *scratch):
    cid = lax.axis_index("c")
    sid = lax.axis_index("s")

    slow_first = SLOW_CID == 0
    base_slow = sid * NPT_SLOW if slow_first else \
        NSUB * NPT_FAST + sid * NPT_SLOW
    base_fast = NSUB * NPT_SLOW + sid * NPT_FAST if slow_first else \
        sid * NPT_FAST

    is_slow = cid == SLOW_CID
    node_base = jnp.where(is_slow, base_slow, base_fast)
    ngroups = jnp.where(is_slow, NPT_SLOW // GROUP, NPT_FAST // GROUP)
    _worker(h_hbm, ei_hbm, out_hbm, *scratch,
            node_base=node_base, ngroups=ngroups)


@functools.partial(
    pl.kernel,
    out_type=jax.ShapeDtypeStruct((N_PAD, C_OUT), jnp.float32),
    mesh=plsc.VectorSubcoreMesh(core_axis_name="c", subcore_axis_name="s"),
    scratch_types=[
        pltpu.VMEM((GMAX * 2, GROW // 2), jnp.int32),
        pltpu.VMEM((GROW, C_OUT), jnp.float32),
        pltpu.VMEM((GROW, C_OUT), jnp.float32),
        pltpu.VMEM((GROUP, C_OUT), jnp.float32),
        pltpu.VMEM((GROUP, C_OUT), jnp.float32),
        pltpu.VMEM((GROUP, C_OUT), jnp.float32),
        pltpu.VMEM((GROUP, C_OUT), jnp.float32),
        pltpu.SemaphoreType.DMA,
        pltpu.SemaphoreType.DMA,
        pltpu.SemaphoreType.DMA,
        pltpu.SemaphoreType.DMA,
        pltpu.SemaphoreType.DMA,
        pltpu.SemaphoreType.DMA,
    ],
)
def _sc_aggregate(h_hbm, ei_hbm, out_hbm, *scratch):
    _sc_body(h_hbm, ei_hbm, out_hbm, *scratch)


def kernel(x, edge_index, W, bias):
    x2 = x[0, :, :, 0]
    x_pad = jnp.pad(x2, ((0, 0), (0, N_PAD - N)))
    w_scaled = W * jnp.float32(1.0 / DEG)
    b_scaled = (bias[0, :, 0, 0] * jnp.float32(1.0 / DEG)).reshape(1, C_OUT)

    h = _compute_h(x_pad, w_scaled, b_scaled)

    ei = edge_index[0, 0].astype(jnp.int32)
    ei_pad = jnp.pad(ei, ((0, N_PAD - N), (0, 0)))
    ei_groups = ei_pad.reshape(N_PAD * K // 128, 128)
    ei_groups = jnp.pad(ei_groups, ((0, GMAX * 2), (0, 0)))

    out_pad = _sc_aggregate(h, ei_groups)

    out = out_pad[:N].T
    return out.reshape(1, C_OUT, N, 1)


# speedup vs baseline: 1.0017x; 1.0017x over previous
import functools

import jax
import jax.numpy as jnp
from jax import lax
from jax.experimental import pallas as pl
from jax.experimental.pallas import tpu as pltpu
from jax.experimental.pallas import tpu_sc as plsc

B, C_IN, C_OUT, N, K = 1, 128, 128, 10000, 32
DEG = K + 1

NSUB = 16
NPT_SLOW = 128
NPT_FAST = 512
SLOW_CID = 1
N_PAD = NSUB * (NPT_SLOW + NPT_FAST)
GROUP = 4
GROW = GROUP * K
NBUF = 4
LANES = 16
CVECS = C_OUT // LANES
GMAX = NPT_FAST // GROUP
TC_BLK = 1024


def _tc_body(x_ref, w_ref, b_ref, o_ref):
    acc = lax.dot_general(
        x_ref[:, :], w_ref[:, :],
        (((0,), (1,)), ((), ())),
        preferred_element_type=jnp.float32,
    )
    o_ref[:, :] = jnp.maximum(acc, 0.0) + b_ref[:, :]


def _compute_h(x_pad, w_scaled, b_scaled):
    return pl.pallas_call(
        _tc_body,
        grid=(N_PAD // TC_BLK,),
        in_specs=[
            pl.BlockSpec((C_IN, TC_BLK), lambda i: (0, i)),
            pl.BlockSpec((C_OUT, C_IN), lambda i: (0, 0)),
            pl.BlockSpec((1, C_OUT), lambda i: (0, 0)),
        ],
        out_specs=pl.BlockSpec((TC_BLK, C_OUT), lambda i: (i, 0)),
        out_shape=jax.ShapeDtypeStruct((N_PAD, C_OUT), jnp.float32),
    )(x_pad, w_scaled, b_scaled)


def _worker(h_hbm, ei_hbm, out_hbm, idx_v,
            rows0_v, rows1_v, rows2_v, rows3_v,
            self0_v, self1_v, self2_v, self3_v,
            out0_v, out1_v, out2_v, out3_v,
            gsem0, gsem1, gsem2, gsem3,
            ssem0, ssem1, ssem2, ssem3,
            osem0, osem1, osem2, osem3,
            node_base, ngroups):
    node_base = pl.multiple_of(node_base, 8)
    rowbase = pl.multiple_of(node_base // GROUP, 8)

    rows = (rows0_v, rows1_v, rows2_v, rows3_v)
    selfs = (self0_v, self1_v, self2_v, self3_v)
    outs = (out0_v, out1_v, out2_v, out3_v)
    gsems = (gsem0, gsem1, gsem2, gsem3)
    ssems = (ssem0, ssem1, ssem2, ssem3)
    osems = (osem0, osem1, osem2, osem3)

    pltpu.sync_copy(ei_hbm.at[pl.ds(rowbase, GMAX)], idx_v)

    def gather(g, b):
        pltpu.async_copy(h_hbm.at[idx_v.at[g]], rows[b], gsems[b])
        pltpu.async_copy(
            h_hbm.at[pl.ds(node_base + g * GROUP, GROUP)], selfs[b],
            ssems[b])

    def wait_gather(g, b):
        pltpu.make_async_copy(h_hbm.at[idx_v.at[g]], rows[b],
                              gsems[b]).wait()
        pltpu.make_async_copy(
            h_hbm.at[pl.ds(node_base + g * GROUP, GROUP)], selfs[b],
            ssems[b]).wait()

    def out_dma(g, b):
        pltpu.async_copy(
            outs[b], out_hbm.at[pl.ds(node_base + g * GROUP, GROUP)],
            osems[b])

    def wait_out(g, b):
        pltpu.make_async_copy(
            outs[b], out_hbm.at[pl.ds(node_base + g * GROUP, GROUP)],
            osems[b]).wait()

    def consume(g, b):
        rows_b = rows[b]
        self_b = selfs[b]
        out_b = outs[b]

        def node(n_, carry):
            base = n_ * K
            for c in range(CVECS):
                cs = pl.ds(c * LANES, LANES)
                s = self_b[n_, cs]
                for r in range(K):
                    s = s + rows_b[base + r, cs]
                out_b[n_, cs] = s
            return carry

        lax.fori_loop(0, GROUP, node, 0)

    for b in range(NBUF):
        gather(b, b)

    def body(o, carry):
        for b in range(NBUF):
            g = o * NBUF + b
            wait_gather(g, b)

            @pl.when(o > 0)
            def _():
                wait_out(g - NBUF, b)

            consume(g, b)
            out_dma(g, b)
            gather(g + NBUF, b)
        return carry

    lax.fori_loop(0, ngroups // NBUF - 1, body, 0)

    g0 = ngroups - NBUF
    for b in range(NBUF):
        g = g0 + b
        wait_gather(g, b)

        @pl.when(g0 > 0)
        def _():
            wait_out(g - NBUF, b)

        consume(g, b)
        out_dma(g, b)
    for b in range(NBUF):
        wait_out(g0 + b, b)


def _sc_body(h_hbm, ei_hbm, out_hbm, *scratch):
    cid = lax.axis_index("c")
    sid = lax.axis_index("s")

    slow_first = SLOW_CID == 0
    base_slow = sid * NPT_SLOW if slow_first else \
        NSUB * NPT_FAST + sid * NPT_SLOW
    base_fast = NSUB * NPT_SLOW + sid * NPT_FAST if slow_first else \
        sid * NPT_FAST

    is_slow = cid == SLOW_CID
    node_base = jnp.where(is_slow, base_slow, base_fast)
    ngroups = jnp.where(is_slow, NPT_SLOW // GROUP, NPT_FAST // GROUP)
    _worker(h_hbm, ei_hbm, out_hbm, *scratch,
            node_base=node_base, ngroups=ngroups)


@functools.partial(
    pl.kernel,
    out_type=jax.ShapeDtypeStruct((N_PAD, C_OUT), jnp.float32),
    mesh=plsc.VectorSubcoreMesh(core_axis_name="c", subcore_axis_name="s"),
    scratch_types=[
        pltpu.VMEM((GMAX, GROW), jnp.int32),
    ] + [pltpu.VMEM((GROW, C_OUT), jnp.float32)] * NBUF
      + [pltpu.VMEM((GROUP, C_OUT), jnp.float32)] * (2 * NBUF)
      + [pltpu.SemaphoreType.DMA] * (3 * NBUF),
)
def _sc_aggregate(h_hbm, ei_hbm, out_hbm, *scratch):
    _sc_body(h_hbm, ei_hbm, out_hbm, *scratch)


def kernel(x, edge_index, W, bias):
    x2 = x[0, :, :, 0]
    x_pad = jnp.pad(x2, ((0, 0), (0, N_PAD - N)))
    w_scaled = W * jnp.float32(1.0 / DEG)
    b_scaled = (bias[0, :, 0, 0] * jnp.float32(1.0 / DEG)).reshape(1, C_OUT)

    h = _compute_h(x_pad, w_scaled, b_scaled)

    ei = edge_index[0, 0].astype(jnp.int32)
    ei_pad = jnp.pad(ei, ((0, N_PAD - N), (0, 0)))
    ei_groups = ei_pad.reshape(N_PAD // GROUP, GROW)
    ei_groups = jnp.pad(ei_groups, ((0, GMAX), (0, 0)))

    out_pad = _sc_aggregate(h, ei_groups)

    out = out_pad[:N].T
    return out.reshape(1, C_OUT, N, 1)


# speedup vs baseline: 1.0050x; 1.0033x over previous
import functools

import jax
import jax.numpy as jnp
from jax import lax
from jax.experimental import pallas as pl
from jax.experimental.pallas import tpu as pltpu
from jax.experimental.pallas import tpu_sc as plsc

B, C_IN, C_OUT, N, K = 1, 128, 128, 10000, 32
DEG = K + 1

NSUB = 16
NPT_SLOW = 128
NPT_FAST = 512
SLOW_CID = 1
N_PAD = NSUB * (NPT_SLOW + NPT_FAST)
GROUP = 4
GROW = GROUP * K
NBUF = 4
SUPER = 8
LANES = 16
CVECS = C_OUT // LANES
GMAX = NPT_FAST // GROUP
TC_BLK = 1024


def _tc_body(x_ref, w_ref, b_ref, o_ref):
    acc = lax.dot_general(
        x_ref[:, :], w_ref[:, :],
        (((0,), (1,)), ((), ())),
        preferred_element_type=jnp.float32,
    )
    o_ref[:, :] = jnp.maximum(acc, 0.0) + b_ref[:, :]


def _compute_h(x_pad, w_scaled, b_scaled):
    return pl.pallas_call(
        _tc_body,
        grid=(N_PAD // TC_BLK,),
        in_specs=[
            pl.BlockSpec((C_IN, TC_BLK), lambda i: (0, i)),
            pl.BlockSpec((C_OUT, C_IN), lambda i: (0, 0)),
            pl.BlockSpec((1, C_OUT), lambda i: (0, 0)),
        ],
        out_specs=pl.BlockSpec((TC_BLK, C_OUT), lambda i: (i, 0)),
        out_shape=jax.ShapeDtypeStruct((N_PAD, C_OUT), jnp.float32),
    )(x_pad, w_scaled, b_scaled)


def _worker(h_hbm, ei_hbm, out_hbm, idx_v,
            rows0_v, rows1_v, rows2_v, rows3_v,
            self0_v, self1_v, out0_v, out1_v,
            gsem0, gsem1, gsem2, gsem3,
            ssem0, ssem1, osem0, osem1,
            node_base, ngroups):
    node_base = pl.multiple_of(node_base, 8)
    rowbase = pl.multiple_of(node_base // GROUP, 8)
    nsuper2 = ngroups // (2 * SUPER)

    rows = (rows0_v, rows1_v, rows2_v, rows3_v)
    selfs = (self0_v, self1_v)
    outs = (out0_v, out1_v)
    gsems = (gsem0, gsem1, gsem2, gsem3)
    ssems = (ssem0, ssem1)
    osems = (osem0, osem1)

    pltpu.sync_copy(ei_hbm.at[pl.ds(rowbase, GMAX)], idx_v)

    SNODES = SUPER * GROUP

    def gather(g, b):
        pltpu.async_copy(h_hbm.at[idx_v.at[g]], rows[b], gsems[b])

    def wait_gather(g, b):
        pltpu.make_async_copy(h_hbm.at[idx_v.at[g]], rows[b],
                              gsems[b]).wait()

    def self_dma(o, sb):
        pltpu.async_copy(
            h_hbm.at[pl.ds(node_base + o * SNODES, SNODES)], selfs[sb],
            ssems[sb])

    def wait_self(o, sb):
        pltpu.make_async_copy(
            h_hbm.at[pl.ds(node_base + o * SNODES, SNODES)], selfs[sb],
            ssems[sb]).wait()

    def out_dma(o, sb):
        pltpu.async_copy(
            outs[sb], out_hbm.at[pl.ds(node_base + o * SNODES, SNODES)],
            osems[sb])

    def wait_out(o, sb):
        pltpu.make_async_copy(
            outs[sb], out_hbm.at[pl.ds(node_base + o * SNODES, SNODES)],
            osems[sb]).wait()

    def consume(gg, sb, b):
        rows_b = rows[b]
        self_b = selfs[sb]
        out_b = outs[sb]

        def node(n_, carry):
            orow = gg * GROUP + n_
            base = n_ * K
            for c in range(CVECS):
                cs = pl.ds(c * LANES, LANES)
                s = self_b[orow, cs]
                for r in range(K):
                    s = s + rows_b[base + r, cs]
                out_b[orow, cs] = s
            return carry

        lax.fori_loop(0, GROUP, node, 0)

    self_dma(0, 0)
    self_dma(1, 1)
    for b in range(NBUF):
        gather(b, b)

    def body(o2, carry):
        for sb in range(2):
            o = o2 * 2 + sb
            wait_self(o, sb)

            @pl.when(o >= 2)
            def _():
                wait_out(o - 2, sb)

            def quad(gg2, carry):
                for b in range(NBUF):
                    gg = gg2 * NBUF + b
                    g = o * SUPER + gg
                    wait_gather(g, b)
                    consume(gg, sb, b)

                    @pl.when(g + NBUF < ngroups)
                    def _():
                        gather(g + NBUF, b)
                return carry

            lax.fori_loop(0, SUPER // NBUF, quad, 0)

            out_dma(o, sb)

            @pl.when(o + 2 < 2 * nsuper2)
            def _():
                self_dma(o + 2, sb)
        return carry

    lax.fori_loop(0, nsuper2, body, 0)

    wait_out(2 * nsuper2 - 2, 0)
    wait_out(2 * nsuper2 - 1, 1)


def _sc_body(h_hbm, ei_hbm, out_hbm, *scratch):
    cid = lax.axis_index("c")
    sid = lax.axis_index("s")

    slow_first = SLOW_CID == 0
    base_slow = sid * NPT_SLOW if slow_first else \
        NSUB * NPT_FAST + sid * NPT_SLOW
    base_fast = NSUB * NPT_SLOW + sid * NPT_FAST if slow_first else \
        sid * NPT_FAST

    is_slow = cid == SLOW_CID
    node_base = jnp.where(is_slow, base_slow, base_fast)
    ngroups = jnp.where(is_slow, NPT_SLOW // GROUP, NPT_FAST // GROUP)
    _worker(h_hbm, ei_hbm, out_hbm, *scratch,
            node_base=node_base, ngroups=ngroups)


@functools.partial(
    pl.kernel,
    out_type=jax.ShapeDtypeStruct((N_PAD, C_OUT), jnp.float32),
    mesh=plsc.VectorSubcoreMesh(core_axis_name="c", subcore_axis_name="s"),
    scratch_types=[
        pltpu.VMEM((GMAX, GROW), jnp.int32),
    ] + [pltpu.VMEM((GROW, C_OUT), jnp.float32)] * NBUF
      + [pltpu.VMEM((SUPER * GROUP, C_OUT), jnp.float32)] * 4
      + [pltpu.SemaphoreType.DMA] * (NBUF + 4),
)
def _sc_aggregate(h_hbm, ei_hbm, out_hbm, *scratch):
    _sc_body(h_hbm, ei_hbm, out_hbm, *scratch)


def kernel(x, edge_index, W, bias):
    x2 = x[0, :, :, 0]
    x_pad = jnp.pad(x2, ((0, 0), (0, N_PAD - N)))
    w_scaled = W * jnp.float32(1.0 / DEG)
    b_scaled = (bias[0, :, 0, 0] * jnp.float32(1.0 / DEG)).reshape(1, C_OUT)

    h = _compute_h(x_pad, w_scaled, b_scaled)

    ei = edge_index[0, 0].astype(jnp.int32)
    ei_pad = jnp.pad(ei, ((0, N_PAD - N), (0, 0)))
    ei_groups = ei_pad.reshape(N_PAD // GROUP, GROW)
    ei_groups = jnp.pad(ei_groups, ((0, GMAX), (0, 0)))

    out_pad = _sc_aggregate(h, ei_groups)

    out = out_pad[:N].T
    return out.reshape(1, C_OUT, N, 1)
